# baseline (speedup 1.0000x reference)
import functools
import jax
import jax.numpy as jnp
from jax import lax
from jax.experimental import pallas as pl
from jax.experimental.pallas import tpu as pltpu
from jax.experimental.pallas import tpu_sc as plsc

N = 100000
D = 128
NCLS = 3
BLK = 5000
GRID = N // BLK

NC = 2
NS = 16
NW = NC * NS
CHUNK = 400
BATCH = 80
NBATCH = CHUNK // BATCH
NCHUNKS = N // CHUNK
KMAX = -(-NCHUNKS // NW)


def _sc_segsum_body(feat_hbm, lab_hbm, out_hbm, chunk_v, lab_v, idx_v,
                    zero_v, shared):
    s = lax.axis_index("s")
    c = lax.axis_index("c")
    wid = s * NC + c

    for i in range(NCLS):
        for j in range(D // 16):
            zero_v[i, pl.ds(16 * j, 16)] = jnp.zeros((16,), jnp.float32)
    pltpu.sync_copy(zero_v, shared.at[pl.ds(NCLS * s, NCLS)])

    def step(k, carry):
        ci = k * NW + wid

        @pl.when(ci < NCHUNKS)
        def _():
            base = ci * CHUNK
            pltpu.sync_copy(feat_hbm.at[pl.ds(base, CHUNK)], chunk_v)
            pltpu.sync_copy(lab_hbm.at[pl.ds(base, CHUNK)], lab_v)
            off = NCLS * s
            for j in range(NBATCH):
                for i in range(BATCH // 16):
                    sl = pl.ds(j * BATCH + 16 * i, 16)
                    idx_v[j, pl.ds(16 * i, 16)] = lab_v[sl] + off
            for j in range(NBATCH):
                pltpu.sync_copy(
                    chunk_v.at[pl.ds(j * BATCH, BATCH)],
                    shared.at[idx_v.at[j]],
                    add=True,
                )
        return carry

    lax.fori_loop(0, KMAX, step, 0)
    pltpu.sync_copy(shared.at[pl.ds(NCLS * s, NCLS)], out_hbm.at[wid])


@functools.partial(
    pl.kernel,
    out_type=jax.ShapeDtypeStruct((NW, NCLS, D), jnp.float32),
    mesh=plsc.VectorSubcoreMesh(core_axis_name="c", subcore_axis_name="s"),
    scratch_types=[
        pltpu.VMEM((CHUNK, D), jnp.float32),
        pltpu.VMEM((CHUNK,), jnp.int32),
        pltpu.VMEM((NBATCH, BATCH), jnp.int32),
        pltpu.VMEM((NCLS, D), jnp.float32),
        pltpu.VMEM_SHARED((NS * NCLS, D), jnp.float32),
    ],
)
def _sc_segsum(feat_hbm, lab_hbm, out_hbm, chunk_v, lab_v, idx_v, zero_v,
               shared):
    _sc_segsum_body(feat_hbm, lab_hbm, out_hbm, chunk_v, lab_v, idx_v,
                    zero_v, shared)


def _dense_body(seq_ref, w_ref, part_ref, out_ref):
    x = seq_ref[...]
    t = x * w_ref[...]
    r = jnp.where(t > 0, t, jnp.exp(t) - 1.0)

    seg = jnp.sum(part_ref[...], axis=0)
    ave = seg * jnp.float32(1.0 / (N // 2))
    an = jnp.sqrt(jnp.sum(ave * ave, axis=1, keepdims=True))
    an = jnp.maximum(an, 1e-8)
    avn = ave / an

    a = lax.dot_general(
        avn, r, (((1,), (1,)), ((), ())),
        preferred_element_type=jnp.float32,
    )
    rr = lax.dot_general(
        jnp.ones((8, D), jnp.float32), r * r, (((1,), (1,)), ((), ())),
        preferred_element_type=jnp.float32,
    )[0:1, :]
    inv_rn = 1.0 / jnp.maximum(jnp.sqrt(rr), 1e-8)
    cos = a * inv_rn

    c0 = cos[0:1, :]
    c1 = cos[1:2, :]
    c2 = cos[2:3, :]
    m = jnp.maximum(jnp.maximum(c0, c1), c2)
    e0 = jnp.exp(c0 - m)
    e1 = jnp.exp(c1 - m)
    e2 = jnp.exp(c2 - m)
    inv_s = 1.0 / (e0 + e1 + e2)
    out_ref[0, 0:1, :] = e0 * inv_s
    out_ref[0, 1:2, :] = e1 * inv_s
    out_ref[0, 2:3, :] = e2 * inv_s


def _dense(seq, weight, partials):
    return pl.pallas_call(
        _dense_body,
        grid=(GRID,),
        in_specs=[
            pl.BlockSpec((BLK, D), lambda i: (i, 0)),
            pl.BlockSpec((1, D), lambda i: (0, 0)),
            pl.BlockSpec((NW, NCLS, D), lambda i: (0, 0, 0)),
        ],
        out_specs=pl.BlockSpec((1, NCLS, BLK), lambda i: (i, 0, 0)),
        out_shape=jax.ShapeDtypeStruct((GRID, NCLS, BLK), jnp.float32),
    )(seq, weight, partials)


@jax.jit
def kernel(seq, feature, labels, weight):
    partials = _sc_segsum(feature, labels)
    out = _dense(seq, weight, partials)
    return out.transpose(0, 2, 1).reshape(N, NCLS)


# speedup vs baseline: 1.0655x; 1.0655x over previous
import functools
import jax
import jax.numpy as jnp
from jax import lax
from jax.experimental import pallas as pl
from jax.experimental.pallas import tpu as pltpu
from jax.experimental.pallas import tpu_sc as plsc

N = 100000
D = 128
NCLS = 3
BLK = 10000
GRID = N // BLK

NC = 2
NS = 16
NW = NC * NS
CHUNK = 400
BATCH = 80
NBATCH = CHUNK // BATCH
NCHUNKS = N // CHUNK
KMAX = -(-NCHUNKS // NW)


def _sc_segsum_body(feat_hbm, lab_hbm, out_hbm, chunk_v, lab_v, idx_v,
                    zero_v, shared):
    s = lax.axis_index("s")
    c = lax.axis_index("c")
    wid = s * NC + c

    for i in range(NCLS):
        for j in range(D // 16):
            zero_v[i, pl.ds(16 * j, 16)] = jnp.zeros((16,), jnp.float32)
    pltpu.sync_copy(zero_v, shared.at[pl.ds(NCLS * s, NCLS)])

    def step(k, carry):
        ci = k * NW + wid

        @pl.when(ci < NCHUNKS)
        def _():
            base = ci * CHUNK
            pltpu.sync_copy(feat_hbm.at[pl.ds(base, CHUNK)], chunk_v)
            pltpu.sync_copy(lab_hbm.at[pl.ds(base, CHUNK)], lab_v)
            off = NCLS * s
            for j in range(NBATCH):
                for i in range(BATCH // 16):
                    sl = pl.ds(j * BATCH + 16 * i, 16)
                    idx_v[j, pl.ds(16 * i, 16)] = lab_v[sl] + off
            for j in range(NBATCH):
                pltpu.sync_copy(
                    chunk_v.at[pl.ds(j * BATCH, BATCH)],
                    shared.at[idx_v.at[j]],
                    add=True,
                )
        return carry

    lax.fori_loop(0, KMAX, step, 0)
    pltpu.sync_copy(shared.at[pl.ds(NCLS * s, NCLS)], out_hbm.at[wid])


@functools.partial(
    pl.kernel,
    out_type=jax.ShapeDtypeStruct((NW, NCLS, D), jnp.float32),
    mesh=plsc.VectorSubcoreMesh(core_axis_name="c", subcore_axis_name="s"),
    scratch_types=[
        pltpu.VMEM((CHUNK, D), jnp.float32),
        pltpu.VMEM((CHUNK,), jnp.int32),
        pltpu.VMEM((NBATCH, BATCH), jnp.int32),
        pltpu.VMEM((NCLS, D), jnp.float32),
        pltpu.VMEM_SHARED((NS * NCLS, D), jnp.float32),
    ],
)
def _sc_segsum(feat_hbm, lab_hbm, out_hbm, chunk_v, lab_v, idx_v, zero_v,
               shared):
    _sc_segsum_body(feat_hbm, lab_hbm, out_hbm, chunk_v, lab_v, idx_v,
                    zero_v, shared)


def _dense_body(seq_ref, w_ref, part_ref, out_ref):
    x = seq_ref[...]
    t = x * w_ref[...]
    r = jnp.where(t > 0, t, jnp.exp(t) - 1.0)

    seg = jnp.sum(part_ref[...], axis=0)
    ave = seg * jnp.float32(1.0 / (N // 2))
    an = jnp.sqrt(jnp.sum(ave * ave, axis=1, keepdims=True))
    an = jnp.maximum(an, 1e-8)
    avn = ave / an

    a = lax.dot_general(
        avn, r, (((1,), (1,)), ((), ())),
        preferred_element_type=jnp.float32,
    )
    rr = lax.dot_general(
        jnp.ones((8, D), jnp.float32), r * r, (((1,), (1,)), ((), ())),
        preferred_element_type=jnp.float32,
    )[0:1, :]
    inv_rn = 1.0 / jnp.maximum(jnp.sqrt(rr), 1e-8)
    cos = a * inv_rn

    c0 = cos[0:1, :]
    c1 = cos[1:2, :]
    c2 = cos[2:3, :]
    m = jnp.maximum(jnp.maximum(c0, c1), c2)
    e0 = jnp.exp(c0 - m)
    e1 = jnp.exp(c1 - m)
    e2 = jnp.exp(c2 - m)
    inv_s = 1.0 / (e0 + e1 + e2)
    out_ref[0, 0:1, :] = e0 * inv_s
    out_ref[0, 1:2, :] = e1 * inv_s
    out_ref[0, 2:3, :] = e2 * inv_s


def _dense(seq, weight, partials):
    return pl.pallas_call(
        _dense_body,
        grid=(GRID,),
        in_specs=[
            pl.BlockSpec((BLK, D), lambda i: (i, 0)),
            pl.BlockSpec((1, D), lambda i: (0, 0)),
            pl.BlockSpec((NW, NCLS, D), lambda i: (0, 0, 0)),
        ],
        out_specs=pl.BlockSpec((1, NCLS, BLK), lambda i: (i, 0, 0)),
        out_shape=jax.ShapeDtypeStruct((GRID, NCLS, BLK), jnp.float32),
    )(seq, weight, partials)


@jax.jit
def kernel(seq, feature, labels, weight):
    partials = _sc_segsum(feature, labels)
    out = _dense(seq, weight, partials)
    return out.transpose(0, 2, 1).reshape(N, NCLS)


# speedup vs baseline: 1.2495x; 1.1726x over previous
import functools
import jax
import jax.numpy as jnp
from jax import lax
from jax.experimental import pallas as pl
from jax.experimental.pallas import tpu as pltpu
from jax.experimental.pallas import tpu_sc as plsc

N = 100000
D = 128
NCLS = 3
BLK = 10000
GRID = N // BLK

NC = 2
NS = 16
NW = NC * NS
CHUNK = 400
BATCH = 80
NBATCH = CHUNK // BATCH
NCHUNKS = N // CHUNK
KMAX = -(-NCHUNKS // NW)


def _sc_segsum_body(feat_hbm, lab_hbm, out_hbm, ch0, ch1, lb0, lb1, idx_v,
                    zero_v, shared, sem0, sem1):
    s = lax.axis_index("s")
    c = lax.axis_index("c")
    wid = s * NC + c
    chunk_bufs = (ch0, ch1)
    lab_bufs = (lb0, lb1)
    sems = (sem0, sem1)

    for i in range(NCLS):
        for j in range(D // 16):
            zero_v[i, pl.ds(16 * j, 16)] = jnp.zeros((16,), jnp.float32)
    pltpu.sync_copy(zero_v, shared.at[pl.ds(NCLS * s, NCLS)])

    def start(k):
        b = k % 2
        ci = k * NW + wid

        @pl.when(ci < NCHUNKS)
        def _():
            base = ci * CHUNK
            pltpu.async_copy(feat_hbm.at[pl.ds(base, CHUNK)], chunk_bufs[b],
                             sems[b])
            pltpu.async_copy(lab_hbm.at[pl.ds(base, CHUNK)], lab_bufs[b],
                             sems[b])

    def finish(k):
        b = k % 2
        ci = k * NW + wid

        @pl.when(ci < NCHUNKS)
        def _():
            pltpu.make_async_copy(feat_hbm.at[pl.ds(0, CHUNK)],
                                  chunk_bufs[b], sems[b]).wait()
            pltpu.make_async_copy(lab_hbm.at[pl.ds(0, CHUNK)],
                                  lab_bufs[b], sems[b]).wait()
            off = NCLS * s
            for j in range(NBATCH):
                for i in range(BATCH // 16):
                    sl = pl.ds(j * BATCH + 16 * i, 16)
                    idx_v[j, pl.ds(16 * i, 16)] = lab_bufs[b][sl] + off
            for j in range(NBATCH):
                pltpu.sync_copy(
                    chunk_bufs[b].at[pl.ds(j * BATCH, BATCH)],
                    shared.at[idx_v.at[j]],
                    add=True,
                )

    start(0)
    for k in range(KMAX):
        if k + 1 < KMAX:
            start(k + 1)
        finish(k)
    pltpu.sync_copy(shared.at[pl.ds(NCLS * s, NCLS)], out_hbm.at[wid])


@functools.partial(
    pl.kernel,
    out_type=jax.ShapeDtypeStruct((NW, NCLS, D), jnp.float32),
    mesh=plsc.VectorSubcoreMesh(core_axis_name="c", subcore_axis_name="s"),
    scratch_types=[
        pltpu.VMEM((CHUNK, D), jnp.float32),
        pltpu.VMEM((CHUNK, D), jnp.float32),
        pltpu.VMEM((CHUNK,), jnp.int32),
        pltpu.VMEM((CHUNK,), jnp.int32),
        pltpu.VMEM((NBATCH, BATCH), jnp.int32),
        pltpu.VMEM((NCLS, D), jnp.float32),
        pltpu.VMEM_SHARED((NS * NCLS, D), jnp.float32),
        pltpu.SemaphoreType.DMA,
        pltpu.SemaphoreType.DMA,
    ],
)
def _sc_segsum(feat_hbm, lab_hbm, out_hbm, ch0, ch1, lb0, lb1, idx_v, zero_v,
               shared, sem0, sem1):
    _sc_segsum_body(feat_hbm, lab_hbm, out_hbm, ch0, ch1, lb0, lb1, idx_v,
                    zero_v, shared, sem0, sem1)


def _dense_body(seq_ref, w_ref, part_ref, out_ref):
    x = seq_ref[...]
    t = x * w_ref[...]
    r = jnp.where(t > 0, t, jnp.exp(t) - 1.0)

    seg = jnp.sum(part_ref[...], axis=0)
    ave = seg * jnp.float32(1.0 / (N // 2))
    an = jnp.sqrt(jnp.sum(ave * ave, axis=1, keepdims=True))
    an = jnp.maximum(an, 1e-8)
    avn = ave / an

    a = lax.dot_general(
        avn, r, (((1,), (1,)), ((), ())),
        preferred_element_type=jnp.float32,
    )
    rr = lax.dot_general(
        jnp.ones((8, D), jnp.float32), r * r, (((1,), (1,)), ((), ())),
        preferred_element_type=jnp.float32,
    )[0:1, :]
    inv_rn = 1.0 / jnp.maximum(jnp.sqrt(rr), 1e-8)
    cos = a * inv_rn

    c0 = cos[0:1, :]
    c1 = cos[1:2, :]
    c2 = cos[2:3, :]
    m = jnp.maximum(jnp.maximum(c0, c1), c2)
    e0 = jnp.exp(c0 - m)
    e1 = jnp.exp(c1 - m)
    e2 = jnp.exp(c2 - m)
    inv_s = 1.0 / (e0 + e1 + e2)
    out_ref[0, 0:1, :] = e0 * inv_s
    out_ref[0, 1:2, :] = e1 * inv_s
    out_ref[0, 2:3, :] = e2 * inv_s


def _dense(seq, weight, partials):
    return pl.pallas_call(
        _dense_body,
        grid=(GRID,),
        in_specs=[
            pl.BlockSpec((BLK, D), lambda i: (i, 0)),
            pl.BlockSpec((1, D), lambda i: (0, 0)),
            pl.BlockSpec((NW, NCLS, D), lambda i: (0, 0, 0)),
        ],
        out_specs=pl.BlockSpec((1, NCLS, BLK), lambda i: (i, 0, 0)),
        out_shape=jax.ShapeDtypeStruct((GRID, NCLS, BLK), jnp.float32),
    )(seq, weight, partials)


@jax.jit
def kernel(seq, feature, labels, weight):
    partials = _sc_segsum(feature, labels)
    out = _dense(seq, weight, partials)
    return out.transpose(0, 2, 1).reshape(N, NCLS)


# speedup vs baseline: 1.2762x; 1.0214x over previous
import functools
import jax
import jax.numpy as jnp
from jax import lax
from jax.experimental import pallas as pl
from jax.experimental.pallas import tpu as pltpu
from jax.experimental.pallas import tpu_sc as plsc

N = 100000
D = 128
NCLS = 3
BLK = 10000
GRID = N // BLK

NC = 2
NS = 16
NW = NC * NS
CHUNK = 400
BATCH = 80
NBATCH = CHUNK // BATCH
NCHUNKS = N // CHUNK
KMAX = -(-NCHUNKS // NW)


def _sc_segsum_body(feat_hbm, lab_hbm, out_hbm, ch0, ch1, lb0, lb1, idx_v,
                    zero_v, shared, sem0, sem1, ssem):
    s = lax.axis_index("s")
    c = lax.axis_index("c")
    wid = s * NC + c
    chunk_bufs = (ch0, ch1)
    lab_bufs = (lb0, lb1)
    sems = (sem0, sem1)

    for i in range(NCLS):
        for j in range(D // 16):
            zero_v[i, pl.ds(16 * j, 16)] = jnp.zeros((16,), jnp.float32)
    pltpu.sync_copy(zero_v, shared.at[pl.ds(NCLS * s, NCLS)])

    def start(k, b):
        ci = k * NW + wid

        @pl.when(ci < NCHUNKS)
        def _():
            base = ci * CHUNK
            pltpu.async_copy(feat_hbm.at[pl.ds(base, CHUNK)], chunk_bufs[b],
                             sems[b])
            pltpu.async_copy(lab_hbm.at[pl.ds(base, CHUNK)], lab_bufs[b],
                             sems[b])

    def finish(k, b):
        ci = k * NW + wid

        @pl.when(ci < NCHUNKS)
        def _():
            pltpu.make_async_copy(feat_hbm.at[pl.ds(0, CHUNK)],
                                  chunk_bufs[b], sems[b]).wait()
            pltpu.make_async_copy(lab_hbm.at[pl.ds(0, CHUNK)],
                                  lab_bufs[b], sems[b]).wait()
            off = NCLS * s
            for j in range(NBATCH):
                for i in range(BATCH // 16):
                    sl = pl.ds(j * BATCH + 16 * i, 16)
                    idx_v[j, pl.ds(16 * i, 16)] = lab_bufs[b][sl] + off
            descs = [
                pltpu.async_copy(
                    chunk_bufs[b].at[pl.ds(j * BATCH, BATCH)],
                    shared.at[idx_v.at[j]],
                    ssem,
                    add=True,
                )
                for j in range(NBATCH)
            ]
            for d in descs:
                d.wait()

    start(0, 0)

    def body(i, carry):
        k0 = 2 * i
        start(k0 + 1, 1)
        finish(k0, 0)
        start(k0 + 2, 0)
        finish(k0 + 1, 1)
        return carry

    lax.fori_loop(0, KMAX // 2, body, 0)
    pltpu.sync_copy(shared.at[pl.ds(NCLS * s, NCLS)], out_hbm.at[wid])


@functools.partial(
    pl.kernel,
    out_type=jax.ShapeDtypeStruct((NW, NCLS, D), jnp.float32),
    mesh=plsc.VectorSubcoreMesh(core_axis_name="c", subcore_axis_name="s"),
    scratch_types=[
        pltpu.VMEM((CHUNK, D), jnp.float32),
        pltpu.VMEM((CHUNK, D), jnp.float32),
        pltpu.VMEM((CHUNK,), jnp.int32),
        pltpu.VMEM((CHUNK,), jnp.int32),
        pltpu.VMEM((NBATCH, BATCH), jnp.int32),
        pltpu.VMEM((NCLS, D), jnp.float32),
        pltpu.VMEM_SHARED((NS * NCLS, D), jnp.float32),
        pltpu.SemaphoreType.DMA,
        pltpu.SemaphoreType.DMA,
        pltpu.SemaphoreType.DMA,
    ],
)
def _sc_segsum(feat_hbm, lab_hbm, out_hbm, ch0, ch1, lb0, lb1, idx_v, zero_v,
               shared, sem0, sem1, ssem):
    _sc_segsum_body(feat_hbm, lab_hbm, out_hbm, ch0, ch1, lb0, lb1, idx_v,
                    zero_v, shared, sem0, sem1, ssem)


def _dense_body(seq_ref, w_ref, part_ref, out_ref):
    x = seq_ref[...]
    t = x * w_ref[...]
    r = jnp.where(t > 0, t, jnp.exp(t) - 1.0)

    seg = jnp.sum(part_ref[...], axis=0)
    ave = seg * jnp.float32(1.0 / (N // 2))
    an = jnp.sqrt(jnp.sum(ave * ave, axis=1, keepdims=True))
    an = jnp.maximum(an, 1e-8)
    avn = ave / an

    a = lax.dot_general(
        avn, r, (((1,), (1,)), ((), ())),
        preferred_element_type=jnp.float32,
    )
    rr = lax.dot_general(
        jnp.ones((8, D), jnp.float32), r * r, (((1,), (1,)), ((), ())),
        preferred_element_type=jnp.float32,
    )[0:1, :]
    inv_rn = 1.0 / jnp.maximum(jnp.sqrt(rr), 1e-8)
    cos = a * inv_rn

    c0 = cos[0:1, :]
    c1 = cos[1:2, :]
    c2 = cos[2:3, :]
    m = jnp.maximum(jnp.maximum(c0, c1), c2)
    e0 = jnp.exp(c0 - m)
    e1 = jnp.exp(c1 - m)
    e2 = jnp.exp(c2 - m)
    inv_s = 1.0 / (e0 + e1 + e2)
    out_ref[0, 0:1, :] = e0 * inv_s
    out_ref[0, 1:2, :] = e1 * inv_s
    out_ref[0, 2:3, :] = e2 * inv_s


def _dense(seq, weight, partials):
    return pl.pallas_call(
        _dense_body,
        grid=(GRID,),
        in_specs=[
            pl.BlockSpec((BLK, D), lambda i: (i, 0)),
            pl.BlockSpec((1, D), lambda i: (0, 0)),
            pl.BlockSpec((NW, NCLS, D), lambda i: (0, 0, 0)),
        ],
        out_specs=pl.BlockSpec((1, NCLS, BLK), lambda i: (i, 0, 0)),
        out_shape=jax.ShapeDtypeStruct((GRID, NCLS, BLK), jnp.float32),
    )(seq, weight, partials)


@jax.jit
def kernel(seq, feature, labels, weight):
    partials = _sc_segsum(feature, labels)
    out = _dense(seq, weight, partials)
    return out.transpose(0, 2, 1).reshape(N, NCLS)


# speedup vs baseline: 1.4321x; 1.1222x over previous
import functools
import jax
import jax.numpy as jnp
from jax import lax
from jax.experimental import pallas as pl
from jax.experimental.pallas import tpu as pltpu
from jax.experimental.pallas import tpu_sc as plsc

N = 100000
D = 128
NCLS = 3
BLK = 10000
GRID = N // BLK

NC = 2
NS = 16
NW = NC * NS
CHUNK = 400
BATCH = 80
NBATCH = CHUNK // BATCH
NCHUNKS = N // CHUNK

TROWS = 50000
SBLK = 5000
TGRID = TROWS // SBLK
CPAD = 8
CBASE = TROWS // CHUNK
KMAX_RAW = -(-(NCHUNKS - CBASE) // NW)
KMAX = KMAX_RAW + (KMAX_RAW % 2)


def _sc_segsum_body(feat_hbm, lab_hbm, out_hbm, ch0, ch1, lb0, lb1, idx_v,
                    zero_v, shared, sem0, sem1, ssem):
    s = lax.axis_index("s")
    c = lax.axis_index("c")
    wid = s * NC + c
    chunk_bufs = (ch0, ch1)
    lab_bufs = (lb0, lb1)
    sems = (sem0, sem1)

    for i in range(NCLS):
        for j in range(D // 16):
            zero_v[i, pl.ds(16 * j, 16)] = jnp.zeros((16,), jnp.float32)
    pltpu.sync_copy(zero_v, shared.at[pl.ds(NCLS * s, NCLS)])

    def start(k, b):
        ci = CBASE + k * NW + wid

        @pl.when(ci < NCHUNKS)
        def _():
            base = ci * CHUNK
            pltpu.async_copy(feat_hbm.at[pl.ds(base, CHUNK)], chunk_bufs[b],
                             sems[b])
            pltpu.async_copy(lab_hbm.at[pl.ds(base, CHUNK)], lab_bufs[b],
                             sems[b])

    def finish(k, b):
        ci = CBASE + k * NW + wid

        @pl.when(ci < NCHUNKS)
        def _():
            pltpu.make_async_copy(feat_hbm.at[pl.ds(0, CHUNK)],
                                  chunk_bufs[b], sems[b]).wait()
            pltpu.make_async_copy(lab_hbm.at[pl.ds(0, CHUNK)],
                                  lab_bufs[b], sems[b]).wait()
            off = NCLS * s
            for j in range(NBATCH):
                for i in range(BATCH // 16):
                    sl = pl.ds(j * BATCH + 16 * i, 16)
                    idx_v[j, pl.ds(16 * i, 16)] = lab_bufs[b][sl] + off
            descs = [
                pltpu.async_copy(
                    chunk_bufs[b].at[pl.ds(j * BATCH, BATCH)],
                    shared.at[idx_v.at[j]],
                    ssem,
                    add=True,
                )
                for j in range(NBATCH)
            ]
            for d in descs:
                d.wait()

    start(0, 0)

    def body(i, carry):
        k0 = 2 * i
        start(k0 + 1, 1)
        finish(k0, 0)
        start(k0 + 2, 0)
        finish(k0 + 1, 1)
        return carry

    lax.fori_loop(0, KMAX // 2, body, 0)
    pltpu.sync_copy(shared.at[pl.ds(NCLS * s, NCLS)], out_hbm.at[wid])


@functools.partial(
    pl.kernel,
    out_type=jax.ShapeDtypeStruct((NW, NCLS, D), jnp.float32),
    mesh=plsc.VectorSubcoreMesh(core_axis_name="c", subcore_axis_name="s"),
    scratch_types=[
        pltpu.VMEM((CHUNK, D), jnp.float32),
        pltpu.VMEM((CHUNK, D), jnp.float32),
        pltpu.VMEM((CHUNK,), jnp.int32),
        pltpu.VMEM((CHUNK,), jnp.int32),
        pltpu.VMEM((NBATCH, BATCH), jnp.int32),
        pltpu.VMEM((NCLS, D), jnp.float32),
        pltpu.VMEM_SHARED((NS * NCLS, D), jnp.float32),
        pltpu.SemaphoreType.DMA,
        pltpu.SemaphoreType.DMA,
        pltpu.SemaphoreType.DMA,
    ],
)
def _sc_segsum(feat_hbm, lab_hbm, out_hbm, ch0, ch1, lb0, lb1, idx_v, zero_v,
               shared, sem0, sem1, ssem):
    _sc_segsum_body(feat_hbm, lab_hbm, out_hbm, ch0, ch1, lb0, lb1, idx_v,
                    zero_v, shared, sem0, sem1, ssem)


def _tc_segsum_body(labels_ref, feat_ref, out_ref):
    step = pl.program_id(0)
    lab = labels_ref[0, 0, :]
    cls = lax.broadcasted_iota(jnp.int32, (CPAD, SBLK), 0)
    onehot = (cls == lab[None, :]).astype(jnp.float32)
    acc = lax.dot_general(
        onehot, feat_ref[...],
        (((1,), (0,)), ((), ())),
        preferred_element_type=jnp.float32,
    )

    @pl.when(step == 0)
    def _():
        out_ref[...] = acc

    @pl.when(step != 0)
    def _():
        out_ref[...] += acc


def _tc_segsum(feature, labels):
    labels3 = labels.reshape(N // SBLK, 1, SBLK)
    return pl.pallas_call(
        _tc_segsum_body,
        grid=(TGRID,),
        in_specs=[
            pl.BlockSpec((1, 1, SBLK), lambda i: (i, 0, 0)),
            pl.BlockSpec((SBLK, D), lambda i: (i, 0)),
        ],
        out_specs=pl.BlockSpec((CPAD, D), lambda i: (0, 0)),
        out_shape=jax.ShapeDtypeStruct((CPAD, D), jnp.float32),
    )(labels3, feature)


def _dense_body(seq_ref, w_ref, part_ref, tcp_ref, out_ref):
    x = seq_ref[...]
    t = x * w_ref[...]
    r = jnp.where(t > 0, t, jnp.exp(t) - 1.0)

    seg = jnp.sum(part_ref[...], axis=0)
    seg = seg + tcp_ref[0:NCLS, :]
    ave = seg * jnp.float32(1.0 / (N // 2))
    an = jnp.sqrt(jnp.sum(ave * ave, axis=1, keepdims=True))
    an = jnp.maximum(an, 1e-8)
    avn = ave / an

    a = lax.dot_general(
        avn, r, (((1,), (1,)), ((), ())),
        preferred_element_type=jnp.float32,
    )
    rr = lax.dot_general(
        jnp.ones((8, D), jnp.float32), r * r, (((1,), (1,)), ((), ())),
        preferred_element_type=jnp.float32,
    )[0:1, :]
    inv_rn = 1.0 / jnp.maximum(jnp.sqrt(rr), 1e-8)
    cos = a * inv_rn

    c0 = cos[0:1, :]
    c1 = cos[1:2, :]
    c2 = cos[2:3, :]
    m = jnp.maximum(jnp.maximum(c0, c1), c2)
    e0 = jnp.exp(c0 - m)
    e1 = jnp.exp(c1 - m)
    e2 = jnp.exp(c2 - m)
    inv_s = 1.0 / (e0 + e1 + e2)
    out_ref[0, 0:1, :] = e0 * inv_s
    out_ref[0, 1:2, :] = e1 * inv_s
    out_ref[0, 2:3, :] = e2 * inv_s


def _dense(seq, weight, partials, tc_partial):
    return pl.pallas_call(
        _dense_body,
        grid=(GRID,),
        in_specs=[
            pl.BlockSpec((BLK, D), lambda i: (i, 0)),
            pl.BlockSpec((1, D), lambda i: (0, 0)),
            pl.BlockSpec((NW, NCLS, D), lambda i: (0, 0, 0)),
            pl.BlockSpec((CPAD, D), lambda i: (0, 0)),
        ],
        out_specs=pl.BlockSpec((1, NCLS, BLK), lambda i: (i, 0, 0)),
        out_shape=jax.ShapeDtypeStruct((GRID, NCLS, BLK), jnp.float32),
    )(seq, weight, partials, tc_partial)


@jax.jit
def kernel(seq, feature, labels, weight):
    partials = _sc_segsum(feature, labels)
    tc_partial = _tc_segsum(feature, labels)
    out = _dense(seq, weight, partials, tc_partial)
    return out.transpose(0, 2, 1).reshape(N, NCLS)


# speedup vs baseline: 1.4422x; 1.0071x over previous
import functools
import jax
import jax.numpy as jnp
from jax import lax
from jax.experimental import pallas as pl
from jax.experimental.pallas import tpu as pltpu
from jax.experimental.pallas import tpu_sc as plsc

N = 100000
D = 128
NCLS = 3
BLK = 10000
GRID = N // BLK

NC = 2
NS = 16
NW = NC * NS
CHUNK = 400
BATCH = 80
NBATCH = CHUNK // BATCH
NCHUNKS = N // CHUNK

TROWS = 60000
SBLK = 5000
TGRID = TROWS // SBLK
CPAD = 8
CBASE = TROWS // CHUNK
KMAX_RAW = -(-(NCHUNKS - CBASE) // NW)
KMAX = KMAX_RAW + (KMAX_RAW % 2)


def _sc_segsum_body(feat_hbm, lab_hbm, out_hbm, ch0, ch1, lb0, lb1, idx_v,
                    zero_v, shared, sem0, sem1, ssem):
    s = lax.axis_index("s")
    c = lax.axis_index("c")
    wid = s * NC + c
    chunk_bufs = (ch0, ch1)
    lab_bufs = (lb0, lb1)
    sems = (sem0, sem1)

    for i in range(NCLS):
        for j in range(D // 16):
            zero_v[i, pl.ds(16 * j, 16)] = jnp.zeros((16,), jnp.float32)
    pltpu.sync_copy(zero_v, shared.at[pl.ds(NCLS * s, NCLS)])

    def start(k, b):
        ci = CBASE + k * NW + wid

        @pl.when(ci < NCHUNKS)
        def _():
            base = ci * CHUNK
            pltpu.async_copy(feat_hbm.at[pl.ds(base, CHUNK)], chunk_bufs[b],
                             sems[b])
            pltpu.async_copy(lab_hbm.at[pl.ds(base, CHUNK)], lab_bufs[b],
                             sems[b])

    def finish(k, b):
        ci = CBASE + k * NW + wid

        @pl.when(ci < NCHUNKS)
        def _():
            pltpu.make_async_copy(feat_hbm.at[pl.ds(0, CHUNK)],
                                  chunk_bufs[b], sems[b]).wait()
            pltpu.make_async_copy(lab_hbm.at[pl.ds(0, CHUNK)],
                                  lab_bufs[b], sems[b]).wait()
            off = NCLS * s
            for j in range(NBATCH):
                for i in range(BATCH // 16):
                    sl = pl.ds(j * BATCH + 16 * i, 16)
                    idx_v[j, pl.ds(16 * i, 16)] = lab_bufs[b][sl] + off
            descs = [
                pltpu.async_copy(
                    chunk_bufs[b].at[pl.ds(j * BATCH, BATCH)],
                    shared.at[idx_v.at[j]],
                    ssem,
                    add=True,
                )
                for j in range(NBATCH)
            ]
            for d in descs:
                d.wait()

    start(0, 0)

    def body(i, carry):
        k0 = 2 * i
        start(k0 + 1, 1)
        finish(k0, 0)
        start(k0 + 2, 0)
        finish(k0 + 1, 1)
        return carry

    lax.fori_loop(0, KMAX // 2, body, 0)
    pltpu.sync_copy(shared.at[pl.ds(NCLS * s, NCLS)], out_hbm.at[wid])


@functools.partial(
    pl.kernel,
    out_type=jax.ShapeDtypeStruct((NW, NCLS, D), jnp.float32),
    mesh=plsc.VectorSubcoreMesh(core_axis_name="c", subcore_axis_name="s"),
    scratch_types=[
        pltpu.VMEM((CHUNK, D), jnp.float32),
        pltpu.VMEM((CHUNK, D), jnp.float32),
        pltpu.VMEM((CHUNK,), jnp.int32),
        pltpu.VMEM((CHUNK,), jnp.int32),
        pltpu.VMEM((NBATCH, BATCH), jnp.int32),
        pltpu.VMEM((NCLS, D), jnp.float32),
        pltpu.VMEM_SHARED((NS * NCLS, D), jnp.float32),
        pltpu.SemaphoreType.DMA,
        pltpu.SemaphoreType.DMA,
        pltpu.SemaphoreType.DMA,
    ],
)
def _sc_segsum(feat_hbm, lab_hbm, out_hbm, ch0, ch1, lb0, lb1, idx_v, zero_v,
               shared, sem0, sem1, ssem):
    _sc_segsum_body(feat_hbm, lab_hbm, out_hbm, ch0, ch1, lb0, lb1, idx_v,
                    zero_v, shared, sem0, sem1, ssem)


def _tc_segsum_body(labels_ref, feat_ref, out_ref):
    step = pl.program_id(0)
    lab = labels_ref[0, 0, :]
    cls = lax.broadcasted_iota(jnp.int32, (CPAD, SBLK), 0)
    onehot = (cls == lab[None, :]).astype(jnp.float32)
    acc = lax.dot_general(
        onehot, feat_ref[...],
        (((1,), (0,)), ((), ())),
        preferred_element_type=jnp.float32,
    )

    @pl.when(step == 0)
    def _():
        out_ref[...] = acc

    @pl.when(step != 0)
    def _():
        out_ref[...] += acc


def _tc_segsum(feature, labels):
    labels3 = labels.reshape(N // SBLK, 1, SBLK)
    return pl.pallas_call(
        _tc_segsum_body,
        grid=(TGRID,),
        in_specs=[
            pl.BlockSpec((1, 1, SBLK), lambda i: (i, 0, 0)),
            pl.BlockSpec((SBLK, D), lambda i: (i, 0)),
        ],
        out_specs=pl.BlockSpec((CPAD, D), lambda i: (0, 0)),
        out_shape=jax.ShapeDtypeStruct((CPAD, D), jnp.float32),
    )(labels3, feature)


def _dense_body(seq_ref, w_ref, part_ref, tcp_ref, out_ref):
    x = seq_ref[...]
    t = x * w_ref[...]
    r = jnp.where(t > 0, t, jnp.exp(t) - 1.0)

    seg = jnp.sum(part_ref[...], axis=0)
    seg = seg + tcp_ref[0:NCLS, :]
    ave = seg * jnp.float32(1.0 / (N // 2))
    an = jnp.sqrt(jnp.sum(ave * ave, axis=1, keepdims=True))
    an = jnp.maximum(an, 1e-8)
    avn = ave / an

    a = lax.dot_general(
        avn, r, (((1,), (1,)), ((), ())),
        preferred_element_type=jnp.float32,
    )
    rr = lax.dot_general(
        jnp.ones((8, D), jnp.float32), r * r, (((1,), (1,)), ((), ())),
        preferred_element_type=jnp.float32,
    )[0:1, :]
    inv_rn = 1.0 / jnp.maximum(jnp.sqrt(rr), 1e-8)
    cos = a * inv_rn

    m = jnp.max(cos, axis=0, keepdims=True)
    e = jnp.exp(cos - m)
    inv_s = 1.0 / jnp.sum(e, axis=0, keepdims=True)
    out_ref[0] = e * inv_s


def _dense(seq, weight, partials, tc_partial):
    return pl.pallas_call(
        _dense_body,
        grid=(GRID,),
        in_specs=[
            pl.BlockSpec((BLK, D), lambda i: (i, 0)),
            pl.BlockSpec((1, D), lambda i: (0, 0)),
            pl.BlockSpec((NW, NCLS, D), lambda i: (0, 0, 0)),
            pl.BlockSpec((CPAD, D), lambda i: (0, 0)),
        ],
        out_specs=pl.BlockSpec((1, NCLS, BLK), lambda i: (i, 0, 0)),
        out_shape=jax.ShapeDtypeStruct((GRID, NCLS, BLK), jnp.float32),
    )(seq, weight, partials, tc_partial)


@jax.jit
def kernel(seq, feature, labels, weight):
    partials = _sc_segsum(feature, labels)
    tc_partial = _tc_segsum(feature, labels)
    out = _dense(seq, weight, partials, tc_partial)
    return out.transpose(0, 2, 1).reshape(N, NCLS)


# speedup vs baseline: 1.4675x; 1.0175x over previous
import functools
import jax
import jax.numpy as jnp
from jax import lax
from jax.experimental import pallas as pl
from jax.experimental.pallas import tpu as pltpu
from jax.experimental.pallas import tpu_sc as plsc

N = 100000
D = 128
NCLS = 3
BLK = 20000
GRID = N // BLK

NC = 2
NS = 16
NW = NC * NS
CHUNK = 400
BATCH = 80
NBATCH = CHUNK // BATCH
NCHUNKS = N // CHUNK

TROWS = 60000
SBLK = 5000
TGRID = TROWS // SBLK
CPAD = 8
CBASE = TROWS // CHUNK
KMAX_RAW = -(-(NCHUNKS - CBASE) // NW)
KMAX = KMAX_RAW + (KMAX_RAW % 2)


def _sc_segsum_body(feat_hbm, lab_hbm, out_hbm, ch0, ch1, lb0, lb1, idx_v,
                    zero_v, shared, sem0, sem1, ssem):
    s = lax.axis_index("s")
    c = lax.axis_index("c")
    wid = s * NC + c
    chunk_bufs = (ch0, ch1)
    lab_bufs = (lb0, lb1)
    sems = (sem0, sem1)

    for i in range(NCLS):
        for j in range(D // 16):
            zero_v[i, pl.ds(16 * j, 16)] = jnp.zeros((16,), jnp.float32)
    pltpu.sync_copy(zero_v, shared.at[pl.ds(NCLS * s, NCLS)])

    def start(k, b):
        ci = CBASE + k * NW + wid

        @pl.when(ci < NCHUNKS)
        def _():
            base = ci * CHUNK
            pltpu.async_copy(feat_hbm.at[pl.ds(base, CHUNK)], chunk_bufs[b],
                             sems[b])
            pltpu.async_copy(lab_hbm.at[pl.ds(base, CHUNK)], lab_bufs[b],
                             sems[b])

    def finish(k, b):
        ci = CBASE + k * NW + wid

        @pl.when(ci < NCHUNKS)
        def _():
            pltpu.make_async_copy(feat_hbm.at[pl.ds(0, CHUNK)],
                                  chunk_bufs[b], sems[b]).wait()
            pltpu.make_async_copy(lab_hbm.at[pl.ds(0, CHUNK)],
                                  lab_bufs[b], sems[b]).wait()
            off = NCLS * s
            for j in range(NBATCH):
                for i in range(BATCH // 16):
                    sl = pl.ds(j * BATCH + 16 * i, 16)
                    idx_v[j, pl.ds(16 * i, 16)] = lab_bufs[b][sl] + off
            descs = [
                pltpu.async_copy(
                    chunk_bufs[b].at[pl.ds(j * BATCH, BATCH)],
                    shared.at[idx_v.at[j]],
                    ssem,
                    add=True,
                )
                for j in range(NBATCH)
            ]
            for d in descs:
                d.wait()

    start(0, 0)

    def body(i, carry):
        k0 = 2 * i
        start(k0 + 1, 1)
        finish(k0, 0)
        start(k0 + 2, 0)
        finish(k0 + 1, 1)
        return carry

    lax.fori_loop(0, KMAX // 2, body, 0)
    pltpu.sync_copy(shared.at[pl.ds(NCLS * s, NCLS)], out_hbm.at[wid])


@functools.partial(
    pl.kernel,
    out_type=jax.ShapeDtypeStruct((NW, NCLS, D), jnp.float32),
    mesh=plsc.VectorSubcoreMesh(core_axis_name="c", subcore_axis_name="s"),
    scratch_types=[
        pltpu.VMEM((CHUNK, D), jnp.float32),
        pltpu.VMEM((CHUNK, D), jnp.float32),
        pltpu.VMEM((CHUNK,), jnp.int32),
        pltpu.VMEM((CHUNK,), jnp.int32),
        pltpu.VMEM((NBATCH, BATCH), jnp.int32),
        pltpu.VMEM((NCLS, D), jnp.float32),
        pltpu.VMEM_SHARED((NS * NCLS, D), jnp.float32),
        pltpu.SemaphoreType.DMA,
        pltpu.SemaphoreType.DMA,
        pltpu.SemaphoreType.DMA,
    ],
)
def _sc_segsum(feat_hbm, lab_hbm, out_hbm, ch0, ch1, lb0, lb1, idx_v, zero_v,
               shared, sem0, sem1, ssem):
    _sc_segsum_body(feat_hbm, lab_hbm, out_hbm, ch0, ch1, lb0, lb1, idx_v,
                    zero_v, shared, sem0, sem1, ssem)


def _tc_segsum_body(labels_ref, feat_ref, out_ref):
    step = pl.program_id(0)
    lab = labels_ref[0, 0, :]
    cls = lax.broadcasted_iota(jnp.int32, (CPAD, SBLK), 0)
    onehot = (cls == lab[None, :]).astype(jnp.float32)
    acc = lax.dot_general(
        onehot, feat_ref[...],
        (((1,), (0,)), ((), ())),
        preferred_element_type=jnp.float32,
    )

    @pl.when(step == 0)
    def _():
        out_ref[...] = acc

    @pl.when(step != 0)
    def _():
        out_ref[...] += acc


def _tc_segsum(feature, labels):
    labels3 = labels.reshape(N // SBLK, 1, SBLK)
    return pl.pallas_call(
        _tc_segsum_body,
        grid=(TGRID,),
        in_specs=[
            pl.BlockSpec((1, 1, SBLK), lambda i: (i, 0, 0)),
            pl.BlockSpec((SBLK, D), lambda i: (i, 0)),
        ],
        out_specs=pl.BlockSpec((CPAD, D), lambda i: (0, 0)),
        out_shape=jax.ShapeDtypeStruct((CPAD, D), jnp.float32),
    )(labels3, feature)


def _dense_body(seq_ref, w_ref, part_ref, tcp_ref, out_ref):
    x = seq_ref[...]
    t = x * w_ref[...]
    r = jnp.where(t > 0, t, jnp.exp(t) - 1.0)

    seg = jnp.sum(part_ref[...], axis=0)
    seg = seg + tcp_ref[0:NCLS, :]
    ave = seg * jnp.float32(1.0 / (N // 2))
    an = jnp.sqrt(jnp.sum(ave * ave, axis=1, keepdims=True))
    an = jnp.maximum(an, 1e-8)
    avn = ave / an

    a = lax.dot_general(
        avn, r, (((1,), (1,)), ((), ())),
        preferred_element_type=jnp.float32,
    )
    rr = lax.dot_general(
        jnp.ones((8, D), jnp.float32), r * r, (((1,), (1,)), ((), ())),
        preferred_element_type=jnp.float32,
    )[0:1, :]
    inv_rn = 1.0 / jnp.maximum(jnp.sqrt(rr), 1e-8)
    cos = a * inv_rn

    m = jnp.max(cos, axis=0, keepdims=True)
    e = jnp.exp(cos - m)
    inv_s = 1.0 / jnp.sum(e, axis=0, keepdims=True)
    out_ref[0] = e * inv_s


def _dense(seq, weight, partials, tc_partial):
    return pl.pallas_call(
        _dense_body,
        grid=(GRID,),
        in_specs=[
            pl.BlockSpec((BLK, D), lambda i: (i, 0)),
            pl.BlockSpec((1, D), lambda i: (0, 0)),
            pl.BlockSpec((NW, NCLS, D), lambda i: (0, 0, 0)),
            pl.BlockSpec((CPAD, D), lambda i: (0, 0)),
        ],
        out_specs=pl.BlockSpec((1, NCLS, BLK), lambda i: (i, 0, 0)),
        out_shape=jax.ShapeDtypeStruct((GRID, NCLS, BLK), jnp.float32),
    )(seq, weight, partials, tc_partial)


@jax.jit
def kernel(seq, feature, labels, weight):
    partials = _sc_segsum(feature, labels)
    tc_partial = _tc_segsum(feature, labels)
    out = _dense(seq, weight, partials, tc_partial)
    return out.transpose(0, 2, 1).reshape(N, NCLS)
